# baseline (speedup 1.0000x reference)
import jax
import jax.numpy as jnp
from jax import lax
from jax.experimental import pallas as pl
from jax.experimental.pallas import tpu as pltpu
from jax.experimental.pallas import tpu_sc as plsc

ROWS = 512
HALF = 256
COLS = 511
OUT_LEN = 131072
NB = 32
NQ = ROWS
WIN = 128
N_WIN = 4
BAND = 128


def _body(x_hbm, xt_hbm, out_hbm, blk, obA, obB, carry,
          inA, inB, outA, outB):
    b = lax.axis_index("s") * 2 + lax.axis_index("c")
    iota = lax.iota(jnp.int32, 16)
    zero16 = jnp.full((16,), 0.0, dtype=jnp.float32)
    c127 = jnp.full((16,), WIN - 1, jnp.int32)
    obufs = (obA, obB)
    osems = (outA, outB)
    isems = (inA, inB)

    def in_pairs(w, hf):
        pairs = []
        for band in (hf, hf + 2):
            rows = pl.ds(band * BAND, BAND)
            if w < N_WIN - 1:
                src = x_hbm.at[b, rows, pl.ds(w * WIN, WIN)]
            else:
                src = xt_hbm.at[b, rows, :]
            pairs.append((src, blk.at[rows, :]))
        return pairs

    def issue_in(w, hf):
        for src, dst in in_pairs(w, hf):
            pltpu.async_copy(src, dst, isems[hf])

    def wait_in(w, hf):
        for src, dst in in_pairs(w, hf):
            pltpu.make_async_copy(src, dst, isems[hf]).wait()

    def out_slice(w, hf):
        return out_hbm.at[b, pl.ds(w * WIN, WIN), pl.ds(hf * WIN, WIN)]

    for rb in range(16):
        carry[pl.ds(rb * 16, 16)] = zero16

    issue_in(0, 0)
    issue_in(0, 1)

    for w in range(N_WIN):
        for hf in range(2):
            ob = obufs[hf]
            wait_in(w, hf)
            if w > 0:
                pltpu.make_async_copy(ob, out_slice(w - 1, hf),
                                      osems[hf]).wait()

            rgs = tuple(range(hf * 8, hf * 8 + 8))

            @plsc.parallel_loop(0, 16, unroll=2)
            def _(k):
                rot = (iota + k) & 15
                edge = rot > 0
                for rg in rgs:
                    rows_t = iota + (rg * 16)
                    t = plsc.load_gather(blk, [rows_t, rot])
                    bo = plsc.load_gather(blk, [rows_t + HALF, rot - 1])
                    cv = carry[pl.ds(rg * 16, 16)]
                    v = t + jnp.where(edge, bo, cv)
                    plsc.store_scatter(
                        ob, [rot, rows_t - (hf * WIN)], v)

            @plsc.parallel_loop(16, 128, unroll=2)
            def _(it):
                q0 = (it // 16) * 16
                k = it % 16
                srow = ((iota + k) & 15) + q0
                for rg in rgs:
                    rows_t = iota + (rg * 16)
                    t = plsc.load_gather(blk, [rows_t, srow])
                    bo = plsc.load_gather(blk, [rows_t + HALF, srow - 1])
                    plsc.store_scatter(
                        ob, [srow, rows_t - (hf * WIN)], t + bo)

            if w < N_WIN - 1:
                for rb in rgs:
                    rows_b = iota + (HALF + rb * 16)
                    carry[pl.ds(rb * 16, 16)] = plsc.load_gather(
                        blk, [rows_b, c127])

            pltpu.async_copy(ob, out_slice(w, hf), osems[hf])
            if w < N_WIN - 1:
                issue_in(w + 1, hf)

    for hf in range(2):
        pltpu.make_async_copy(obufs[hf], out_slice(N_WIN - 1, hf),
                              osems[hf]).wait()


@jax.jit
def kernel(x):
    xf = x.reshape(NB, ROWS, COLS)
    xt = jnp.pad(xf[:, :, (N_WIN - 1) * WIN:], ((0, 0), (0, 0), (0, 1)))
    mesh = plsc.VectorSubcoreMesh(core_axis_name="c", subcore_axis_name="s")
    out = pl.kernel(
        _body,
        out_type=jax.ShapeDtypeStruct((NB, NQ, HALF), jnp.float32),
        mesh=mesh,
        scratch_types=[
            pltpu.VMEM((ROWS, WIN), jnp.float32),
            pltpu.VMEM((WIN, WIN), jnp.float32),
            pltpu.VMEM((WIN, WIN), jnp.float32),
            pltpu.VMEM((HALF,), jnp.float32),
            pltpu.SemaphoreType.DMA,
            pltpu.SemaphoreType.DMA,
            pltpu.SemaphoreType.DMA,
            pltpu.SemaphoreType.DMA,
        ],
        compiler_params=pltpu.CompilerParams(
            use_tc_tiling_on_sc=True, needs_layout_passes=False
        ),
    )(xf, xt)
    return out.reshape(*x.shape[:-2], OUT_LEN)
